# Initial kernel scaffold; baseline (speedup 1.0000x reference)
import jax
import jax.numpy as jnp
from jax.experimental import pallas as pl


def kernel(pos, edge_index, sigmas, W_init, b_init, Wl1, bl1, Wr1, br1, We1, att1, bias1, Wl2, bl2, Wr2, br2, We2, att2, bias2, W_p1, b_p1, W_p2, b_p2):
    raise NotImplementedError("write your pallas kernel here")


# speedup vs baseline: 34.8930x; 34.8930x over previous
import functools

import jax
import jax.numpy as jnp
from jax import lax
from jax.experimental import pallas as pl
from jax.experimental.pallas import tpu as pltpu
from jax.experimental.pallas import tpu_sc as plsc

N = 50000
E = 800000
H = 2
C = 16
NG = 50
HC = 32
G = 50
A = 1000

NCORE = 2
NSUB = 16
NTILE = NCORE * NSUB
EP = 802816
GRP = EP // 128
GPT = GRP // NTILE
NP = 50016
RPS = NP // NSUB
DUMP = N
ACLIP = 75.0
SLOPE = 0.2
GSTEP = 5.0 / (NG - 1)
GCOEFF = -0.5 / GSTEP ** 2
NGP = 56

@functools.lru_cache(maxsize=None)
def _mesh():
    return plsc.VectorSubcoreMesh(core_axis_name="c", subcore_axis_name="s",
                                  num_cores=NCORE, num_subcores=NSUB)


def _wid(cid, sid):
    return sid * NCORE + cid


def _iota16():
    return lax.iota(jnp.int32, 16)


def _full16(v):
    return jnp.full((16,), v, jnp.int32)


@functools.lru_cache(maxsize=None)
def _get_k_dist():
    return pl.kernel(
        _k_dist_body,
        out_type=jax.ShapeDtypeStruct((EP,), jnp.float32),
        mesh=_mesh(),
        compiler_params=pltpu.CompilerParams(needs_layout_passes=False, use_tc_tiling_on_sc=False),
        scratch_types=[
            pltpu.VMEM((1792,), jnp.int32),
            pltpu.VMEM((1792,), jnp.int32),
            pltpu.VMEM((1792, 4), jnp.float32),
            pltpu.VMEM((1792, 4), jnp.float32),
            pltpu.VMEM((1792,), jnp.float32),
            pltpu.SemaphoreType.DMA,
        ],
    )


def _k_dist_body(pos4, srcp, dstp, s_out, sidx, didx, prows, qrows, sbuf, sem):
    cid = lax.axis_index("c")
    sid = lax.axis_index("s")
    base = _wid(cid, sid) * (GPT * 128)

    def chunk(k, _):
        off = base + k * 1792
        pltpu.sync_copy(srcp.at[pl.ds(off, 1792)], sidx)
        pltpu.sync_copy(dstp.at[pl.ds(off, 1792)], didx)
        pltpu.async_copy(pos4.at[sidx], prows, sem).wait()
        pltpu.async_copy(pos4.at[didx], qrows, sem).wait()

        def blk(j, _):
            rows = j * 16 + _iota16()
            acc = jnp.zeros((16,), jnp.float32)
            for c3 in range(3):
                cv = _full16(c3)
                xs = plsc.load_gather(prows, [rows, cv])
                xd = plsc.load_gather(qrows, [rows, cv])
                dd = xs - xd
                acc = acc + dd * dd
            sbuf[pl.ds(j * 16, 16)] = acc
            return 0

        lax.fori_loop(0, 112, blk, 0)
        pltpu.sync_copy(sbuf, s_out.at[pl.ds(off, 1792)])
        return 0

    lax.fori_loop(0, 14, chunk, 0)


@functools.lru_cache(maxsize=None)
def _make_segsum():
    def body(et_hbm, dstp, z32, esum_out, acc, didx, etile, payload, sem):
        cid = lax.axis_index("c")
        sid = lax.axis_index("s")
        gbase = _wid(cid, sid) * GPT

        for zz in range(6):
            pltpu.sync_copy(z32, acc.at[pl.ds(sid * RPS + zz * 521, 521)])
        plsc.subcore_barrier()

        def grp(g, _):
            gg = gbase + g
            pltpu.sync_copy(dstp.at[pl.ds(gg * 128, 128)], didx)
            pltpu.sync_copy(et_hbm.at[gg], etile)

            def edge(j, _):
                col0 = plsc.load_gather(etile, [_iota16(), _full16(0) + j])
                col1 = plsc.load_gather(etile, [16 + _iota16(), _full16(0) + j])
                payload[j, pl.ds(0, 16)] = col0
                payload[j, pl.ds(16, 16)] = col1
                return 0

            lax.fori_loop(0, 128, edge, 0)
            pltpu.sync_copy(payload, acc.at[didx], add=True)
            return 0

        lax.fori_loop(0, GPT, grp, 0)
        plsc.subcore_barrier()
        r0 = sid * RPS
        pltpu.sync_copy(acc.at[pl.ds(r0, RPS)], esum_out.at[cid, pl.ds(r0, RPS)])

    return pl.kernel(
        body,
        out_type=jax.ShapeDtypeStruct((NCORE, NP, 32), jnp.float32),
        mesh=_mesh(),
        compiler_params=pltpu.CompilerParams(needs_layout_passes=False, use_tc_tiling_on_sc=False),
        scratch_types=[
            pltpu.VMEM_SHARED((NP, 32), jnp.float32),
            pltpu.VMEM((128,), jnp.int32),
            pltpu.VMEM((32, 128), jnp.float32),
            pltpu.VMEM((128, 32), jnp.float32),
            pltpu.SemaphoreType.DMA,
        ],
    )


@functools.lru_cache(maxsize=None)
def _get_k_layer():
    return pl.kernel(
        _k_layer_body,
        out_type=jax.ShapeDtypeStruct((NCORE, NP, 36), jnp.float32),
        mesh=_mesh(),
        compiler_params=pltpu.CompilerParams(needs_layout_passes=False, use_tc_tiling_on_sc=False),
        scratch_types=[
            pltpu.VMEM_SHARED((NP, 36), jnp.float32),
            pltpu.VMEM((128,), jnp.int32),
            pltpu.VMEM((128,), jnp.int32),
            pltpu.VMEM((32, 128), jnp.float32),
            pltpu.VMEM((128, 32), jnp.float32),
            pltpu.VMEM((128, 32), jnp.float32),
            pltpu.VMEM((128, 36), jnp.float32),
            pltpu.VMEM((32,), jnp.float32),
            pltpu.SemaphoreType.DMA,
            pltpu.SemaphoreType.DMA,
        ],
    )


def _k_layer_body(srcp, dstp, et_hbm, xl_hbm, xr_hbm, att_hbm, z36, lp_out,
             acc, sidx, didx, etile, xlr, xrr, payload, attv, sem1, sem2):
    cid = lax.axis_index("c")
    sid = lax.axis_index("s")
    gbase = _wid(cid, sid) * GPT

    for zz in range(6):
        pltpu.sync_copy(z36, acc.at[pl.ds(sid * RPS + zz * 521, 521)])
    pltpu.sync_copy(att_hbm, attv)
    zero16 = jnp.zeros((16,), jnp.float32)
    one16 = jnp.ones((16,), jnp.float32)
    for j in range(8):
        rows = j * 16 + _iota16()
        plsc.store_scatter(payload, [rows, _full16(34)], one16)
        plsc.store_scatter(payload, [rows, _full16(35)], zero16)
    plsc.subcore_barrier()

    def grp(g, _):
        gg = gbase + g
        pltpu.sync_copy(srcp.at[pl.ds(gg * 128, 128)], sidx)
        pltpu.sync_copy(dstp.at[pl.ds(gg * 128, 128)], didx)
        pltpu.sync_copy(et_hbm.at[gg], etile)
        pltpu.async_copy(xl_hbm.at[sidx], xlr, sem1).wait()
        pltpu.async_copy(xr_hbm.at[didx], xrr, sem2).wait()

        def blk(j, _):
            rows = j * 16 + _iota16()
            for hh in range(2):
                a = jnp.zeros((16,), jnp.float32)
                av = attv[pl.ds(hh * 16, 16)]
                for cc in range(16):
                    col = hh * 16 + cc
                    cv = _full16(col)
                    xv = plsc.load_gather(xlr, [rows, cv])
                    rv = plsc.load_gather(xrr, [rows, cv])
                    ev = plsc.load_gather(etile, [cv, rows])
                    m = xv + rv + ev
                    m = jnp.maximum(m, m * SLOPE)
                    a = a + m * av[cc]
                w = jnp.exp(jnp.minimum(a, ACLIP))
                plsc.store_scatter(payload, [rows, _full16(32 + hh)], w)
                for cc in range(16):
                    col = hh * 16 + cc
                    cv = _full16(col)
                    xv = plsc.load_gather(xlr, [rows, cv])
                    plsc.store_scatter(payload, [rows, cv], xv * w)
            return 0

        lax.fori_loop(0, 8, blk, 0)
        pltpu.sync_copy(payload, acc.at[didx], add=True)
        return 0

    lax.fori_loop(0, GPT, grp, 0)
    plsc.subcore_barrier()
    r0 = sid * RPS
    pltpu.sync_copy(acc.at[pl.ds(r0, RPS)], lp_out.at[cid, pl.ds(r0, RPS)])


def _softplus(x):
    return jnp.maximum(x, 0.0) + jnp.log(1.0 + jnp.exp(-jnp.abs(x)))


RN = 2000


def _tc_node_body(pos_ref, wi_ref, bi_ref, wl_ref, bl_ref, wr_ref, br_ref,
                  xl_ref, xr_ref):
    x0 = _softplus(jnp.dot(pos_ref[...], wi_ref[...],
                           preferred_element_type=jnp.float32) + bi_ref[...])
    xl_ref[...] = jnp.dot(x0, wl_ref[...],
                          preferred_element_type=jnp.float32) + bl_ref[...]
    xr_ref[...] = jnp.dot(x0, wr_ref[...],
                          preferred_element_type=jnp.float32) + br_ref[...]


def _tc_node(pos, W_init, b_init, Wl, bl, Wr, br):
    full = lambda shp: pl.BlockSpec(shp, lambda i: (0,) * len(shp))
    return pl.pallas_call(
        _tc_node_body,
        grid=(N // RN,),
        in_specs=[
            pl.BlockSpec((RN, 3), lambda i: (i, 0)),
            full((3, 16)), full((1, 16)),
            full((16, HC)), full((1, HC)),
            full((16, HC)), full((1, HC)),
        ],
        out_specs=[
            pl.BlockSpec((RN, HC), lambda i: (i, 0)),
            pl.BlockSpec((RN, HC), lambda i: (i, 0)),
        ],
        out_shape=[
            jax.ShapeDtypeStruct((N, HC), jnp.float32),
            jax.ShapeDtypeStruct((N, HC), jnp.float32),
        ],
    )(pos, W_init, b_init.reshape(1, 16), Wl, bl.reshape(1, HC),
      Wr, br.reshape(1, HC))


EGB = 8


def _tc_edge_body(s_ref, w1_ref, w2_ref, e1_ref, e2_ref):
    d = jnp.sqrt(s_ref[...] + 1e-12)
    offc = lax.broadcasted_iota(jnp.int32, (NGP, 1), 0).astype(jnp.float32) * GSTEP
    for g in range(EGB):
        dg = d[g:g + 1, :]
        ea = jnp.exp(GCOEFF * (offc - dg) ** 2)
        e1_ref[g] = jnp.dot(w1_ref[...], ea,
                            preferred_element_type=jnp.float32)
        e2_ref[g] = jnp.dot(w2_ref[...], ea,
                            preferred_element_type=jnp.float32)


def _tc_edge(s2d, We1T, We2T):
    full = lambda shp: pl.BlockSpec(shp, lambda i: (0,) * len(shp))
    return pl.pallas_call(
        _tc_edge_body,
        grid=(GRP // EGB,),
        in_specs=[
            pl.BlockSpec((EGB, 128), lambda i: (i, 0)),
            full((HC, NGP)), full((HC, NGP)),
        ],
        out_specs=[
            pl.BlockSpec((EGB, HC, 128), lambda i: (i, 0, 0)),
            pl.BlockSpec((EGB, HC, 128), lambda i: (i, 0, 0)),
        ],
        out_shape=[
            jax.ShapeDtypeStruct((GRP, HC, 128), jnp.float32),
            jax.ShapeDtypeStruct((GRP, HC, 128), jnp.float32),
        ],
    )(s2d, We1T, We2T)


def _fin_core(l0, l1, s0, s1, c0, c1, xl, xr, attr, biasr):
    num = l0[:, 0:HC] + l1[:, 0:HC]
    den0 = l0[:, 32:33] + l1[:, 32:33]
    den1 = l0[:, 33:34] + l1[:, 33:34]
    esum = s0 + s1
    cnt = c0 + c1
    el = esum / jnp.maximum(cnt, 1.0)
    m = xl + xr + el
    m = jnp.maximum(m, m * SLOPE)
    t = m * attr
    a0 = jnp.sum(t[:, 0:C], axis=1, keepdims=True)
    a1 = jnp.sum(t[:, C:HC], axis=1, keepdims=True)
    w0 = jnp.exp(jnp.minimum(a0, ACLIP))
    w1 = jnp.exp(jnp.minimum(a1, ACLIP))
    num0 = num[:, 0:C] + w0 * xl[:, 0:C]
    num1 = num[:, C:HC] + w1 * xl[:, C:HC]
    den0 = den0 + w0
    den1 = den1 + w1
    out = (num0 / den0 + num1 / den1) * 0.5 / (cnt + 1.0) + biasr
    return _softplus(out)


def _tc_fin_mid_body(l0_ref, l1_ref, s0_ref, s1_ref, c0_ref, c1_ref,
                     xl_ref, xr_ref, att_ref, bias_ref,
                     wl_ref, bl_ref, wr_ref, br_ref, xlo_ref, xro_ref):
    x = _fin_core(l0_ref[...], l1_ref[...], s0_ref[...], s1_ref[...],
                  c0_ref[...], c1_ref[...], xl_ref[...], xr_ref[...],
                  att_ref[...], bias_ref[...])
    xlo_ref[...] = jnp.dot(x, wl_ref[...],
                           preferred_element_type=jnp.float32) + bl_ref[...]
    xro_ref[...] = jnp.dot(x, wr_ref[...],
                           preferred_element_type=jnp.float32) + br_ref[...]


def _tc_fin_mid(lp, es, cnt0, cnt1, xl, xr, att, bias, Wl, bl, Wr, br):
    full = lambda shp: pl.BlockSpec(shp, lambda i: (0,) * len(shp))
    row = lambda w: pl.BlockSpec((RN, w), lambda i: (i, 0))
    return pl.pallas_call(
        _tc_fin_mid_body,
        grid=(N // RN,),
        in_specs=[
            row(36), row(36), row(32), row(32), row(1), row(1),
            row(HC), row(HC), full((1, HC)), full((1, C)),
            full((C, HC)), full((1, HC)), full((C, HC)), full((1, HC)),
        ],
        out_specs=[row(HC), row(HC)],
        out_shape=[
            jax.ShapeDtypeStruct((N, HC), jnp.float32),
            jax.ShapeDtypeStruct((N, HC), jnp.float32),
        ],
    )(lp[0, :N], lp[1, :N], es[0, :N], es[1, :N], cnt0, cnt1,
      xl, xr, att.reshape(1, HC), bias.reshape(1, C),
      Wl, bl.reshape(1, HC), Wr, br.reshape(1, HC))


def _tc_fin_last_body(l0_ref, l1_ref, s0_ref, s1_ref, c0_ref, c1_ref,
                      xl_ref, xr_ref, att_ref, bias_ref,
                      wp1_ref, bp1_ref, wp2_ref, bp2_ref, sig_ref, out_ref):
    x = _fin_core(l0_ref[...], l1_ref[...], s0_ref[...], s1_ref[...],
                  c0_ref[...], c1_ref[...], xl_ref[...], xr_ref[...],
                  att_ref[...], bias_ref[...])
    y = _softplus(jnp.dot(x, wp1_ref[...],
                          preferred_element_type=jnp.float32) + bp1_ref[...])
    sc = jnp.dot(y, wp2_ref[...],
                 preferred_element_type=jnp.float32) + bp2_ref[...]
    out_ref[...] = sc / sig_ref[...]


def _tc_fin_last(lp, es, cnt0, cnt1, xl, xr, att, bias, W_p1, b_p1, W_p2, b_p2, sig):
    full = lambda shp: pl.BlockSpec(shp, lambda i: (0,) * len(shp))
    row = lambda w: pl.BlockSpec((RN, w), lambda i: (i, 0))
    return pl.pallas_call(
        _tc_fin_last_body,
        grid=(N // RN,),
        in_specs=[
            row(36), row(36), row(32), row(32), row(1), row(1),
            row(HC), row(HC), full((1, HC)), full((1, C)),
            full((C, C)), full((1, C)), full((C, 3)), full((1, 3)),
            row(1),
        ],
        out_specs=row(3),
        out_shape=jax.ShapeDtypeStruct((N, 3), jnp.float32),
    )(lp[0, :N], lp[1, :N], es[0, :N], es[1, :N], cnt0, cnt1,
      xl, xr, att.reshape(1, HC), bias.reshape(1, C),
      W_p1, b_p1.reshape(1, C), W_p2, b_p2.reshape(1, 3), sig)


def _padN(x):
    return jnp.pad(x, ((0, NP - N), (0, 0)))


def kernel(pos, edge_index, sigmas, W_init, b_init, Wl1, bl1, Wr1, br1, We1,
           att1, bias1, Wl2, bl2, Wr2, br2, We2, att2, bias2,
           W_p1, b_p1, W_p2, b_p2):
    src = edge_index[0]
    dst = edge_index[1]
    srcp = jnp.concatenate([src, jnp.zeros((EP - E,), jnp.int32)])
    dstp = jnp.concatenate([dst, jnp.full((EP - E,), DUMP, jnp.int32)])
    pos4 = jnp.pad(pos, ((0, NP - N), (0, 1)))
    z32 = jnp.zeros((521, 32), jnp.float32)
    z36 = jnp.zeros((521, 36), jnp.float32)
    We1T = jnp.pad(We1, ((0, NGP - NG), (0, 0))).T
    We2T = jnp.pad(We2, ((0, NGP - NG), (0, 0))).T
    att1v = att1.reshape(HC)
    att2v = att2.reshape(HC)
    sig = sigmas.reshape(N, 1)

    s = _get_k_dist()(pos4, srcp, dstp)
    s2d = s.reshape(GRP, 128)
    e1t, e2t = _tc_edge(s2d, We1T, We2T)
    xl1, xr1 = _tc_node(pos, W_init, b_init, Wl1, bl1, Wr1, br1)

    es1 = _make_segsum()(e1t, dstp, z32)
    es2 = _make_segsum()(e2t, dstp, z32)

    lp1 = _get_k_layer()(srcp, dstp, e1t, _padN(xl1), _padN(xr1), att1v, z36)
    cnt0 = lp1[0, :N, 34:35]
    cnt1 = lp1[1, :N, 34:35]
    xl2, xr2 = _tc_fin_mid(lp1, es1, cnt0, cnt1, xl1, xr1, att1, bias1,
                           Wl2, bl2, Wr2, br2)
    lp2 = _get_k_layer()(srcp, dstp, e2t, _padN(xl2), _padN(xr2), att2v, z36)
    xl3, xr3 = _tc_fin_mid(lp2, es2, cnt0, cnt1, xl2, xr2, att2, bias2,
                           Wl2, bl2, Wr2, br2)
    lp3 = _get_k_layer()(srcp, dstp, e2t, _padN(xl3), _padN(xr3), att2v, z36)
    scores = _tc_fin_last(lp3, es2, cnt0, cnt1, xl3, xr3, att2, bias2,
                          W_p1, b_p1, W_p2, b_p2, sig)
    return scores
